# baseline (speedup 1.0000x reference)
import functools

import jax
import jax.numpy as jnp
from jax.experimental import pallas as pl
from jax.experimental.pallas import tpu as pltpu

_N = 50000
_E = 800000
_G = 64
_H = 16


def _segment_mean_num(feat_e, dst, deg_inv):
    agg = jax.ops.segment_sum(feat_e, dst, num_segments=_N)
    return agg * deg_inv[:, None]


def _conv(feat, src, dst, deg_inv, rel_w, rel_b, root_w):
    fw = feat @ rel_w
    agg = jax.ops.segment_sum(fw[src], dst, num_segments=_N)
    return jax.nn.relu(agg * deg_inv[:, None] + rel_b + feat @ root_w)


def _edge_scores(x2, src, dst, pool_w, pool_b):
    a = x2 @ pool_w[:_H, 0] + pool_b[0]
    b = x2 @ pool_w[_H:, 0]
    e = a[src] + b[dst]
    emax = jax.ops.segment_max(e, dst, num_segments=_N)
    emax = jnp.where(jnp.isfinite(emax), emax, 0.0)
    ex = jnp.exp(e - emax[dst])
    denom = jax.ops.segment_sum(ex, dst, num_segments=_N)
    return ex / (denom[dst] + 1e-16) + 0.5


def _matching(src, dst, rank):
    inf = jnp.int32(2**30)

    def cond(st):
        alive, _, _ = st
        return jnp.any(alive)

    def body(st):
        alive, matched, chosen = st
        r = jnp.where(alive, rank, inf)
        bs = jax.ops.segment_min(r, src, num_segments=_N)
        bt = jax.ops.segment_min(r, dst, num_segments=_N)
        best = jnp.minimum(bs, bt)
        dom = alive & (rank == best[src]) & (rank == best[dst])
        matched = matched.at[jnp.where(dom, src, _N)].set(True, mode="drop")
        matched = matched.at[jnp.where(dom, dst, _N)].set(True, mode="drop")
        alive = alive & ~matched[src] & ~matched[dst]
        return alive, matched, chosen | dom

    alive0 = jnp.ones((_E,), bool)
    matched0 = jnp.zeros((_N,), bool)
    chosen0 = jnp.zeros((_E,), bool)
    alive, matched, chosen = jax.lax.while_loop(
        cond, body, (alive0, matched0, chosen0))
    return chosen, matched


def _global_mean_pool(x, seg):
    s = jax.ops.segment_sum(x, seg, num_segments=_G)
    cnt = jax.ops.segment_sum(jnp.ones((x.shape[0],), x.dtype), seg,
                              num_segments=_G)
    return s / jnp.maximum(cnt, 1.0)[:, None]


def _head_kernel(h1_ref, h2_ref, h3_ref, w1a_ref, w1b_ref, w1c_ref, b1_ref,
                 w2_ref, b2_ref, o_ref):
    h = (jnp.dot(h1_ref[...], w1a_ref[...], preferred_element_type=jnp.float32)
         + jnp.dot(h2_ref[...], w1b_ref[...], preferred_element_type=jnp.float32)
         + jnp.dot(h3_ref[...], w1c_ref[...], preferred_element_type=jnp.float32)
         + b1_ref[...])
    h = jnp.maximum(h, 0.0)
    o = jnp.dot(h, w2_ref[...], preferred_element_type=jnp.float32) + b2_ref[...]
    m = jnp.max(o, axis=-1, keepdims=True)
    lse = jnp.log(jnp.sum(jnp.exp(o - m), axis=-1, keepdims=True)) + m
    o_ref[...] = o - lse


def _dense_head(h1, h2, h3, lin1_w, lin1_b, lin2_w, lin2_b):
    return pl.pallas_call(
        _head_kernel,
        out_shape=jax.ShapeDtypeStruct((_G, 2), jnp.float32),
    )(h1, h2, h3, lin1_w[:_H], lin1_w[_H:2 * _H], lin1_w[2 * _H:],
      lin1_b[None, :], lin2_w, lin2_b[None, :])


def kernel(x, edge_index, batch, c1_rel_w, c1_rel_b, c1_root_w,
           c2_rel_w, c2_rel_b, c2_root_w, c3_rel_w, c3_rel_b, c3_root_w,
           pool_w, pool_b, lin1_w, lin1_b, lin2_w, lin2_b):
    src = edge_index[0]
    dst = edge_index[1]
    i32 = jnp.int32

    deg = jax.ops.segment_sum(jnp.ones((_E,), jnp.float32), dst,
                              num_segments=_N)
    deg_inv = 1.0 / jnp.maximum(deg, 1.0)

    x1 = _conv(x, src, dst, deg_inv, c1_rel_w, c1_rel_b, c1_root_w)
    x2 = _conv(x1, src, dst, deg_inv, c2_rel_w, c2_rel_b, c2_root_w)

    e = _edge_scores(x2, src, dst, pool_w, pool_b)

    perm = jnp.argsort(-e, stable=True)
    rank = jnp.zeros((_E,), i32).at[perm].set(jnp.arange(_E, dtype=i32))
    chosen_e, matched = _matching(src, dst, rank)

    ch_sorted = chosen_e[perm]
    pos = jnp.cumsum(ch_sorted.astype(i32)) - 1
    n_chosen = pos[-1] + 1
    i_edge = pos[rank]
    cluster = jnp.zeros((_N,), i32)
    cluster = cluster.at[jnp.where(chosen_e, src, _N)].set(i_edge, mode="drop")
    cluster = cluster.at[jnp.where(chosen_e, dst, _N)].set(i_edge, mode="drop")
    chosen = jnp.zeros((_N,), i32).at[
        jnp.where(ch_sorted, pos, _N)].set(perm.astype(i32), mode="drop")
    unmatched = ~matched
    csum = jnp.cumsum(unmatched.astype(i32))
    cluster = jnp.where(unmatched, n_chosen + csum - 1, cluster)

    c_src = cluster[src]
    c_dst = cluster[dst]
    key = c_src.astype(jnp.uint32) * jnp.uint32(_N) + c_dst.astype(jnp.uint32)
    ordk = jnp.argsort(key)
    ks = key[ordk]
    first_sorted = jnp.concatenate(
        [jnp.ones((1,), bool), ks[1:] != ks[:-1]])
    first = jnp.zeros((_E,), bool).at[ordk].set(first_sorted)
    cd_eff = jnp.where(first, c_dst, i32(_N))

    enc = jnp.arange(_N, dtype=i32) * _G + batch
    m = jax.ops.segment_max(enc, cluster, num_segments=_N)
    nb = jnp.where(m >= 0, m % _G, i32(_G))

    sv = jnp.where(jnp.arange(_N, dtype=i32) < n_chosen, e[chosen],
                   jnp.ones((_N,), x.dtype))
    new_x = jax.ops.segment_sum(x2, cluster, num_segments=_N) * sv[:, None]

    deg3 = jax.ops.segment_sum(jnp.ones((_E,), jnp.float32), cd_eff,
                               num_segments=_N)
    deg3_inv = 1.0 / jnp.maximum(deg3, 1.0)
    x3 = _conv(new_x, c_src, cd_eff, deg3_inv, c3_rel_w, c3_rel_b, c3_root_w)

    h1 = _global_mean_pool(x1, batch)
    h2 = _global_mean_pool(x2, batch)
    h3 = _global_mean_pool(x3, nb)
    return _dense_head(h1, h2, h3, lin1_w, lin1_b, lin2_w, lin2_b)


# speedup vs baseline: 11.2666x; 11.2666x over previous
import functools

import jax
import jax.numpy as jnp
from jax.experimental import pallas as pl
from jax.experimental.pallas import tpu as pltpu
from jax.experimental.pallas import tpu_sc as plsc

_N = 50000
_E = 800000
_G = 64
_H = 16

_NC = 2
_NS = 16
_NW = _NC * _NS
_NP = 50048
_RPT = _NP // _NS
_EPW = _E // _NW
_CHUNK = 1000
_NCH = _EPW // _CHUNK
_SUB = 8
_SUBW = _CHUNK // _SUB


def _seg_rows_body(table, srcv, dst2d, zeros, out, sidx, didx, rows, shared,
                   sem):
    c = jax.lax.axis_index("c")
    s = jax.lax.axis_index("s")
    wid = s * _NC + c
    pltpu.sync_copy(zeros.at[pl.ds(s * _RPT, _RPT)],
                    shared.at[pl.ds(s * _RPT, _RPT)])
    plsc.subcore_barrier()

    def chunk(i, carry):
        off = wid * _EPW + i * _CHUNK
        row2 = wid * (_EPW // _SUBW) + i * _SUB
        pltpu.sync_copy(srcv.at[pl.ds(off, _CHUNK)], sidx)
        pltpu.sync_copy(dst2d.at[pl.ds(row2, _SUB)], didx)
        pltpu.async_copy(table.at[sidx], rows, sem).wait()
        for j in range(_SUB):
            pltpu.sync_copy(rows.at[pl.ds(j * _SUBW, _SUBW)],
                            shared.at[didx.at[j]], add=True)
        return carry

    jax.lax.fori_loop(0, _NCH, chunk, 0)
    plsc.subcore_barrier()
    pltpu.sync_copy(shared.at[pl.ds(s * _RPT, _RPT)],
                    out.at[c].at[pl.ds(s * _RPT, _RPT)])


_seg_rows = functools.partial(
    pl.kernel,
    out_type=jax.ShapeDtypeStruct((_NC, _NP, _H), jnp.float32),
    mesh=plsc.VectorSubcoreMesh(core_axis_name="c", subcore_axis_name="s"),
    scratch_types=[
        pltpu.VMEM((_CHUNK,), jnp.int32),
        pltpu.VMEM((_SUB, _SUBW), jnp.int32),
        pltpu.VMEM((_CHUNK, _H), jnp.float32),
        pltpu.VMEM_SHARED((_NP, _H), jnp.float32),
        pltpu.SemaphoreType.DMA,
    ],
    compiler_params=pltpu.CompilerParams(use_tc_tiling_on_sc=False),
)(_seg_rows_body)


def _segment_sum_rows(table, src, dst):
    dst2d = dst.reshape(_E // _SUBW, _SUBW)
    zeros = jnp.zeros((_NP, _H), jnp.float32)
    out = _seg_rows(table, src, dst2d, zeros)
    return out[0, :_N] + out[1, :_N]


def _conv(feat, src, dst, deg_inv, rel_w, rel_b, root_w):
    fw = feat @ rel_w
    agg = _segment_sum_rows(fw, src, dst)
    return jax.nn.relu(agg * deg_inv[:, None] + rel_b + feat @ root_w)


_MROWS = _E // 16
_MCH = 250
_MNCH = _MROWS // _MCH


def _match_body(sp2d, dp2d, ok2d, mask_out, sbuf, tbuf, okbuf, mask, probe):
    c = jax.lax.axis_index("c")
    s = jax.lax.axis_index("s")

    @pl.when(jnp.logical_and(c == 0, s == 0))
    def _():
        ones16 = jnp.ones((16,), jnp.int32)

        def initm(i, carry):
            mask[pl.ds(i * 16, 16)] = ones16
            return carry

        jax.lax.fori_loop(0, _N // 16, initm, 0)

        iota = jax.lax.iota(jnp.int32, 16)
        iota16 = iota + 16

        def group(g, carry):
            sv = sbuf[g]
            tv = tbuf[g]
            plsc.store_scatter(probe, [sv], iota)
            plsc.store_scatter(probe, [tv], iota16)
            rs = plsc.load_gather(probe, [sv])
            rt = plsc.load_gather(probe, [tv])
            clean = jnp.logical_and(
                jax.lax.reduce_and(rs == iota, axes=(0,)),
                jax.lax.reduce_and(rt == iota16, axes=(0,)))

            def fast(_):
                ms = plsc.load_gather(mask, [sv])
                mt = plsc.load_gather(mask, [tv])
                ok = jax.lax.mul(ms, mt)
                keep = 1 - ok
                plsc.store_scatter(mask, [sv], jax.lax.mul(ms, keep))
                plsc.store_scatter(mask, [tv], jax.lax.mul(mt, keep))
                return ok

            def slow(_):
                acc = jnp.zeros((16,), jnp.int32)
                for i in range(16):
                    lane = iota == i
                    ms = plsc.load_gather(mask, [sv])
                    mt = plsc.load_gather(mask, [tv])
                    ok = jax.lax.mul(ms, mt)
                    oki = jnp.where(lane, ok, 0)
                    keep = 1 - oki
                    plsc.store_scatter(mask, [sv], jax.lax.mul(ms, keep),
                                       mask=lane)
                    plsc.store_scatter(mask, [tv], jax.lax.mul(mt, keep),
                                       mask=lane)
                    acc = acc + oki
                return acc

            okbuf[g] = jax.lax.cond(clean, fast, slow, 0)
            return carry

        def chunk(ch, carry):
            row0 = ch * _MCH
            pltpu.sync_copy(sp2d.at[pl.ds(row0, _MCH)], sbuf)
            pltpu.sync_copy(dp2d.at[pl.ds(row0, _MCH)], tbuf)
            jax.lax.fori_loop(0, _MCH, group, 0)
            pltpu.sync_copy(okbuf, ok2d.at[pl.ds(row0, _MCH)])
            return carry

        jax.lax.fori_loop(0, _MNCH, chunk, 0)
        pltpu.sync_copy(mask, mask_out)


_match_call = functools.partial(
    pl.kernel,
    out_type=(jax.ShapeDtypeStruct((_MROWS, 16), jnp.int32),
              jax.ShapeDtypeStruct((_N,), jnp.int32)),
    mesh=plsc.VectorSubcoreMesh(core_axis_name="c", subcore_axis_name="s"),
    scratch_types=[
        pltpu.VMEM((_MCH, 16), jnp.int32),
        pltpu.VMEM((_MCH, 16), jnp.int32),
        pltpu.VMEM((_MCH, 16), jnp.int32),
        pltpu.VMEM((_N,), jnp.int32),
        pltpu.VMEM((_N,), jnp.int32),
    ],
    compiler_params=pltpu.CompilerParams(use_tc_tiling_on_sc=False,
                                         needs_layout_passes=False),
)(_match_body)


def _sc_matching(src_p, dst_p):
    ok2d, mask_out = _match_call(src_p.reshape(_MROWS, 16),
                                 dst_p.reshape(_MROWS, 16))
    return ok2d.reshape(_E), mask_out


def _edge_scores(x2, src, dst, pool_w, pool_b):
    a = x2 @ pool_w[:_H, 0] + pool_b[0]
    b = x2 @ pool_w[_H:, 0]
    e = a[src] + b[dst]
    emax = jax.ops.segment_max(e, dst, num_segments=_N)
    emax = jnp.where(jnp.isfinite(emax), emax, 0.0)
    ex = jnp.exp(e - emax[dst])
    denom = jax.ops.segment_sum(ex, dst, num_segments=_N)
    return ex / (denom[dst] + 1e-16) + 0.5


def _global_mean_pool(x, seg):
    s = jax.ops.segment_sum(x, seg, num_segments=_G)
    cnt = jax.ops.segment_sum(jnp.ones((x.shape[0],), x.dtype), seg,
                              num_segments=_G)
    return s / jnp.maximum(cnt, 1.0)[:, None]


def _head_kernel(h1_ref, h2_ref, h3_ref, w1a_ref, w1b_ref, w1c_ref, b1_ref,
                 w2_ref, b2_ref, o_ref):
    h = (jnp.dot(h1_ref[...], w1a_ref[...], preferred_element_type=jnp.float32)
         + jnp.dot(h2_ref[...], w1b_ref[...], preferred_element_type=jnp.float32)
         + jnp.dot(h3_ref[...], w1c_ref[...], preferred_element_type=jnp.float32)
         + b1_ref[...])
    h = jnp.maximum(h, 0.0)
    o = jnp.dot(h, w2_ref[...], preferred_element_type=jnp.float32) + b2_ref[...]
    m = jnp.max(o, axis=-1, keepdims=True)
    lse = jnp.log(jnp.sum(jnp.exp(o - m), axis=-1, keepdims=True)) + m
    o_ref[...] = o - lse


def _dense_head(h1, h2, h3, lin1_w, lin1_b, lin2_w, lin2_b):
    return pl.pallas_call(
        _head_kernel,
        out_shape=jax.ShapeDtypeStruct((_G, 2), jnp.float32),
    )(h1, h2, h3, lin1_w[:_H], lin1_w[_H:2 * _H], lin1_w[2 * _H:],
      lin1_b[None, :], lin2_w, lin2_b[None, :])


def kernel(x, edge_index, batch, c1_rel_w, c1_rel_b, c1_root_w,
           c2_rel_w, c2_rel_b, c2_root_w, c3_rel_w, c3_rel_b, c3_root_w,
           pool_w, pool_b, lin1_w, lin1_b, lin2_w, lin2_b):
    src = edge_index[0]
    dst = edge_index[1]
    i32 = jnp.int32

    deg = jax.ops.segment_sum(jnp.ones((_E,), jnp.float32), dst,
                              num_segments=_N)
    deg_inv = 1.0 / jnp.maximum(deg, 1.0)

    x1 = _conv(x, src, dst, deg_inv, c1_rel_w, c1_rel_b, c1_root_w)
    x2 = _conv(x1, src, dst, deg_inv, c2_rel_w, c2_rel_b, c2_root_w)

    e = _edge_scores(x2, src, dst, pool_w, pool_b)

    perm = jnp.argsort(-e, stable=True)
    src_p = src[perm]
    dst_p = dst[perm]
    ok, free = _sc_matching(src_p, dst_p)
    okb = ok > 0
    pos = jnp.cumsum(ok) - 1
    n_chosen = pos[-1] + 1
    cluster = jnp.zeros((_N,), i32)
    cluster = cluster.at[jnp.where(okb, src_p, _N)].set(pos, mode="drop")
    cluster = cluster.at[jnp.where(okb, dst_p, _N)].set(pos, mode="drop")
    chosen = jnp.zeros((_N,), i32).at[
        jnp.where(okb, pos, _N)].set(perm.astype(i32), mode="drop")
    unmatched = free > 0
    csum = jnp.cumsum(free)
    cluster = jnp.where(unmatched, n_chosen + csum - 1, cluster)

    c_src = cluster[src]
    c_dst = cluster[dst]
    key = c_src.astype(jnp.uint32) * jnp.uint32(_N) + c_dst.astype(jnp.uint32)
    ordk = jnp.argsort(key)
    ks = key[ordk]
    first_sorted = jnp.concatenate(
        [jnp.ones((1,), bool), ks[1:] != ks[:-1]])
    first = jnp.zeros((_E,), bool).at[ordk].set(first_sorted)
    cd_eff = jnp.where(first, c_dst, i32(_N))

    enc = jnp.arange(_N, dtype=i32) * _G + batch
    m = jax.ops.segment_max(enc, cluster, num_segments=_N)
    nb = jnp.where(m >= 0, m % _G, i32(_G))

    sv = jnp.where(jnp.arange(_N, dtype=i32) < n_chosen, e[chosen],
                   jnp.ones((_N,), x.dtype))
    new_x = jax.ops.segment_sum(x2, cluster, num_segments=_N) * sv[:, None]

    deg3 = jax.ops.segment_sum(jnp.ones((_E,), jnp.float32), cd_eff,
                               num_segments=_N)
    deg3_inv = 1.0 / jnp.maximum(deg3, 1.0)
    x3 = _conv(new_x, c_src, cd_eff, deg3_inv, c3_rel_w, c3_rel_b, c3_root_w)

    h1 = _global_mean_pool(x1, batch)
    h2 = _global_mean_pool(x2, batch)
    h3 = _global_mean_pool(x3, nb)
    return _dense_head(h1, h2, h3, lin1_w, lin1_b, lin2_w, lin2_b)
